# Initial kernel scaffold; baseline (speedup 1.0000x reference)
import functools

import jax
import jax.numpy as jnp
from jax import lax
from jax.experimental import pallas as pl
from jax.experimental.pallas import tpu as pltpu
from jax.experimental.pallas import tpu_sc as plsc

N = 10000
E = 320000
D = 128
H = 8
C = 16

NC = 2
NS = 16
NW = NC * NS
EPW = E // NW
EB = 80
ES = EPW // EB
NPAD = 10240
RPT = NPAD // NS

_mesh = plsc.VectorSubcoreMesh(core_axis_name="c", subcore_axis_name="s")


@functools.partial(
    pl.kernel,
    out_type=jax.ShapeDtypeStruct((NC, NPAD), jnp.float32),
    mesh=_mesh,
    scratch_types=[
        pltpu.VMEM((ES, EB), jnp.int32),
        pltpu.VMEM((EB,), jnp.float32),
        pltpu.VMEM((RPT,), jnp.float32),
        pltpu.VMEM_SHARED((NPAD,), jnp.float32),
    ],
)
def _sc_degree(dst_hbm, out_hbm, di_v, ones_v, zb_v, acc):
    c = lax.axis_index("c")
    s = lax.axis_index("s")
    w = c * NS + s

    def fill(i, _):
        zb_v[pl.ds(i * 16, 16)] = jnp.zeros((16,), jnp.float32)
        return 0

    lax.fori_loop(0, RPT // 16, fill, 0)

    def ofill(i, _):
        ones_v[pl.ds(i * 16, 16)] = jnp.ones((16,), jnp.float32)
        return 0

    lax.fori_loop(0, EB // 16, ofill, 0)
    pltpu.sync_copy(zb_v, acc.at[pl.ds(s * RPT, RPT)])
    pltpu.sync_copy(dst_hbm.at[w], di_v)
    plsc.subcore_barrier()

    def step(j, _):
        pltpu.sync_copy(ones_v, acc.at[di_v.at[j]], add=True)
        return 0

    lax.fori_loop(0, ES, step, 0)
    plsc.subcore_barrier()
    pltpu.sync_copy(acc.at[pl.ds(s * RPT, RPT)],
                    out_hbm.at[c, pl.ds(s * RPT, RPT)])


@functools.partial(
    pl.kernel,
    out_type=jax.ShapeDtypeStruct((NC, NPAD, C), jnp.float32),
    mesh=_mesh,
    scratch_types=[
        pltpu.VMEM((ES, EB), jnp.int32),
        pltpu.VMEM((ES, EB), jnp.int32),
        pltpu.VMEM((EB, C), jnp.float32),
        pltpu.VMEM((RPT, C), jnp.float32),
        pltpu.VMEM_SHARED((NPAD, C), jnp.float32),
        pltpu.SemaphoreType.DMA,
    ],
)
def _sc_aggregate(tab_hbm, src_hbm, dst_hbm, out_hbm,
                  si_v, di_v, rows_v, zb_v, acc, sem):
    c = lax.axis_index("c")
    s = lax.axis_index("s")
    w = c * NS + s

    def fill(i, _):
        zb_v[i] = jnp.zeros((C,), jnp.float32)
        return 0

    lax.fori_loop(0, RPT, fill, 0)
    pltpu.sync_copy(zb_v, acc.at[pl.ds(s * RPT, RPT)])
    pltpu.sync_copy(src_hbm.at[w], si_v)
    pltpu.sync_copy(dst_hbm.at[w], di_v)
    plsc.subcore_barrier()

    def step(j, _):
        pltpu.async_copy(tab_hbm.at[si_v.at[j]], rows_v, sem).wait()
        pltpu.sync_copy(rows_v, acc.at[di_v.at[j]], add=True)
        return 0

    lax.fori_loop(0, ES, step, 0)
    plsc.subcore_barrier()
    pltpu.sync_copy(acc.at[pl.ds(s * RPT, RPT)],
                    out_hbm.at[c, pl.ds(s * RPT, RPT)])


def _tc_prep_body(x_ref, w1_ref, degp_ref, xs_ref, dis_ref):
    deg = degp_ref[:N, 0:1] + degp_ref[:N, 1:2] + 1.0
    dis = lax.rsqrt(deg)
    xw = jnp.dot(x_ref[...], w1_ref[...],
                 preferred_element_type=jnp.float32)
    xs = xw * dis
    xs_ref[...] = jnp.concatenate(
        [xs, jnp.zeros((N, C - H), jnp.float32)], axis=1)
    dis_ref[...] = dis


_tc_prep = pl.pallas_call(
    _tc_prep_body,
    out_shape=(jax.ShapeDtypeStruct((N, C), jnp.float32),
               jax.ShapeDtypeStruct((N, 1), jnp.float32)),
)


def _tc_mid_body(aggp_ref, xs_ref, dis_ref, b1_ref, w2_ref, out_ref):
    agg = aggp_ref[0, :N, :] + aggp_ref[1, :N, :] + xs_ref[...]
    t = dis_ref[...] * agg
    h = jnp.maximum(t[:, :H] + b1_ref[...], 0.0)
    hw = jnp.dot(h, w2_ref[...], preferred_element_type=jnp.float32)
    out_ref[...] = hw * dis_ref[...]


_tc_mid = pl.pallas_call(
    _tc_mid_body,
    out_shape=jax.ShapeDtypeStruct((N, C), jnp.float32),
)


def _tc_out_body(aggp_ref, xs2_ref, dis_ref, b2_ref, out_ref):
    o = dis_ref[...] * (aggp_ref[0, :N, :] + aggp_ref[1, :N, :]
                        + xs2_ref[...]) + b2_ref[...]
    m = jnp.max(o, axis=1, keepdims=True)
    e = jnp.exp(o - m)
    lse = jnp.log(jnp.sum(e, axis=1, keepdims=True)) + m
    out_ref[...] = o - lse


_tc_out = pl.pallas_call(
    _tc_out_body,
    out_shape=jax.ShapeDtypeStruct((N, C), jnp.float32),
)


def kernel(x, edge_index, W1, b1, W2, b2):
    src = edge_index[0].reshape(NW, ES, EB)
    dst = edge_index[1].reshape(NW, ES, EB)
    degp = _sc_degree(dst)
    xs1, dis = _tc_prep(x, W1, degp.T)
    agg1 = _sc_aggregate(xs1, src, dst)
    xs2 = _tc_mid(agg1, xs1, dis, b1.reshape(1, H), W2)
    agg2 = _sc_aggregate(xs2, src, dst)
    return _tc_out(agg2, xs2, dis, b2.reshape(1, C))


# speedup vs baseline: 30.5987x; 30.5987x over previous
import functools

import jax
import jax.numpy as jnp
from jax import lax
from jax.experimental import pallas as pl
from jax.experimental.pallas import tpu as pltpu
from jax.experimental.pallas import tpu_sc as plsc

N = 10000
E = 320000
D = 128
H = 8
C = 16

NC = 2
NS = 16
NW = NC * NS
EPW = E // NW
EB = 80
ES = EPW // EB
NPAD = 10240
RPT = NPAD // NS

_mesh = plsc.VectorSubcoreMesh(core_axis_name="c", subcore_axis_name="s")
_sc_params = pltpu.CompilerParams(use_tc_tiling_on_sc=False)


@functools.partial(
    pl.kernel,
    out_type=jax.ShapeDtypeStruct((NC, NPAD), jnp.float32),
    mesh=_mesh,
    compiler_params=_sc_params,
    scratch_types=[
        pltpu.VMEM((ES, EB), jnp.int32),
        pltpu.VMEM((EB,), jnp.float32),
        pltpu.VMEM((RPT,), jnp.float32),
        pltpu.VMEM_SHARED((NPAD,), jnp.float32),
    ],
)
def _sc_degree(dst_hbm, out_hbm, di_v, ones_v, zb_v, acc):
    c = lax.axis_index("c")
    s = lax.axis_index("s")
    w = c * NS + s

    def fill(i, _):
        zb_v[pl.ds(i * 16, 16)] = jnp.zeros((16,), jnp.float32)
        return 0

    lax.fori_loop(0, RPT // 16, fill, 0)

    def ofill(i, _):
        ones_v[pl.ds(i * 16, 16)] = jnp.ones((16,), jnp.float32)
        return 0

    lax.fori_loop(0, EB // 16, ofill, 0)
    pltpu.sync_copy(zb_v, acc.at[pl.ds(s * RPT, RPT)])
    pltpu.sync_copy(dst_hbm.at[w], di_v)
    plsc.subcore_barrier()

    def step(j, _):
        pltpu.sync_copy(ones_v, acc.at[di_v.at[j]], add=True)
        return 0

    lax.fori_loop(0, ES, step, 0)
    plsc.subcore_barrier()
    pltpu.sync_copy(acc.at[pl.ds(s * RPT, RPT)],
                    out_hbm.at[c, pl.ds(s * RPT, RPT)])


@functools.partial(
    pl.kernel,
    out_type=jax.ShapeDtypeStruct((NC, NPAD, C), jnp.float32),
    mesh=_mesh,
    compiler_params=_sc_params,
    scratch_types=[
        pltpu.VMEM((ES, EB), jnp.int32),
        pltpu.VMEM((ES, EB), jnp.int32),
        pltpu.VMEM((EB, C), jnp.float32),
        pltpu.VMEM((RPT, C), jnp.float32),
        pltpu.VMEM_SHARED((NPAD, C), jnp.float32),
        pltpu.SemaphoreType.DMA,
    ],
)
def _sc_aggregate(tab_hbm, src_hbm, dst_hbm, out_hbm,
                  si_v, di_v, rows_v, zb_v, acc, sem):
    c = lax.axis_index("c")
    s = lax.axis_index("s")
    w = c * NS + s

    def fill(i, _):
        zb_v[i] = jnp.zeros((C,), jnp.float32)
        return 0

    lax.fori_loop(0, RPT, fill, 0)
    pltpu.sync_copy(zb_v, acc.at[pl.ds(s * RPT, RPT)])
    pltpu.sync_copy(src_hbm.at[w], si_v)
    pltpu.sync_copy(dst_hbm.at[w], di_v)
    plsc.subcore_barrier()

    def step(j, _):
        pltpu.async_copy(tab_hbm.at[si_v.at[j]], rows_v, sem).wait()
        pltpu.sync_copy(rows_v, acc.at[di_v.at[j]], add=True)
        return 0

    lax.fori_loop(0, ES, step, 0)
    plsc.subcore_barrier()
    pltpu.sync_copy(acc.at[pl.ds(s * RPT, RPT)],
                    out_hbm.at[c, pl.ds(s * RPT, RPT)])


def _tc_prep_body(x_ref, w1_ref, degp_ref, xs_ref, dis_ref):
    deg = degp_ref[:N, 0:1] + degp_ref[:N, 1:2] + 1.0
    dis = lax.rsqrt(deg)
    xw = jnp.dot(x_ref[...], w1_ref[...],
                 preferred_element_type=jnp.float32)
    xs = xw * dis
    xs_ref[...] = jnp.concatenate(
        [xs, jnp.zeros((N, C - H), jnp.float32)], axis=1)
    dis_ref[...] = dis


_tc_prep = pl.pallas_call(
    _tc_prep_body,
    out_shape=(jax.ShapeDtypeStruct((N, C), jnp.float32),
               jax.ShapeDtypeStruct((N, 1), jnp.float32)),
)


def _tc_mid_body(aggp_ref, xs_ref, dis_ref, b1_ref, w2_ref, out_ref):
    agg = aggp_ref[0, :N, :] + aggp_ref[1, :N, :] + xs_ref[...]
    t = dis_ref[...] * agg
    h = jnp.maximum(t[:, :H] + b1_ref[...], 0.0)
    hw = jnp.dot(h, w2_ref[...], preferred_element_type=jnp.float32)
    out_ref[...] = hw * dis_ref[...]


_tc_mid = pl.pallas_call(
    _tc_mid_body,
    out_shape=jax.ShapeDtypeStruct((N, C), jnp.float32),
)


def _tc_out_body(aggp_ref, xs2_ref, dis_ref, b2_ref, out_ref):
    o = dis_ref[...] * (aggp_ref[0, :N, :] + aggp_ref[1, :N, :]
                        + xs2_ref[...]) + b2_ref[...]
    m = jnp.max(o, axis=1, keepdims=True)
    e = jnp.exp(o - m)
    lse = jnp.log(jnp.sum(e, axis=1, keepdims=True)) + m
    out_ref[...] = o - lse


_tc_out = pl.pallas_call(
    _tc_out_body,
    out_shape=jax.ShapeDtypeStruct((N, C), jnp.float32),
)


def kernel(x, edge_index, W1, b1, W2, b2):
    src = edge_index[0].reshape(NW, ES, EB)
    dst = edge_index[1].reshape(NW, ES, EB)
    degp = _sc_degree(dst)
    xs1, dis = _tc_prep(x, W1, degp.T)
    agg1 = _sc_aggregate(xs1, src, dst)
    xs2 = _tc_mid(agg1, xs1, dis, b1.reshape(1, H), W2)
    agg2 = _sc_aggregate(xs2, src, dst)
    return _tc_out(agg2, xs2, dis, b2.reshape(1, C))


# speedup vs baseline: 45.9827x; 1.5028x over previous
import functools

import jax
import jax.numpy as jnp
from jax import lax
from jax.experimental import pallas as pl
from jax.experimental.pallas import tpu as pltpu
from jax.experimental.pallas import tpu_sc as plsc

N = 10000
E = 320000
D = 128
H = 8
C = 16

NC = 2
NS = 16
NW = NC * NS
EPW = E // NW
EB = 80
ES = EPW // EB
NPAD = 10240
RPT = NPAD // NS

_mesh = plsc.VectorSubcoreMesh(core_axis_name="c", subcore_axis_name="s")
_sc_params = pltpu.CompilerParams(use_tc_tiling_on_sc=False)


@functools.partial(
    pl.kernel,
    out_type=jax.ShapeDtypeStruct((NC, NPAD), jnp.float32),
    mesh=_mesh,
    compiler_params=_sc_params,
    scratch_types=[
        pltpu.VMEM((ES, EB), jnp.int32),
        pltpu.VMEM((EB,), jnp.float32),
        pltpu.VMEM((RPT,), jnp.float32),
        pltpu.VMEM_SHARED((NPAD,), jnp.float32),
    ],
)
def _sc_degree(dst_hbm, out_hbm, di_v, ones_v, zb_v, acc):
    c = lax.axis_index("c")
    s = lax.axis_index("s")
    w = c * NS + s

    def fill(i, _):
        zb_v[pl.ds(i * 16, 16)] = jnp.zeros((16,), jnp.float32)
        return 0

    lax.fori_loop(0, RPT // 16, fill, 0)

    def ofill(i, _):
        ones_v[pl.ds(i * 16, 16)] = jnp.ones((16,), jnp.float32)
        return 0

    lax.fori_loop(0, EB // 16, ofill, 0)
    pltpu.sync_copy(zb_v, acc.at[pl.ds(s * RPT, RPT)])
    pltpu.sync_copy(dst_hbm.at[w], di_v)
    plsc.subcore_barrier()

    def step(j, _):
        pltpu.sync_copy(ones_v, acc.at[di_v.at[j]], add=True)
        return 0

    lax.fori_loop(0, ES, step, 0)
    plsc.subcore_barrier()
    pltpu.sync_copy(acc.at[pl.ds(s * RPT, RPT)],
                    out_hbm.at[c, pl.ds(s * RPT, RPT)])


NB = 4


@functools.partial(
    pl.kernel,
    out_type=jax.ShapeDtypeStruct((NC, NPAD, C), jnp.float32),
    mesh=_mesh,
    compiler_params=_sc_params,
    scratch_types=[
        pltpu.VMEM((ES, EB), jnp.int32),
        pltpu.VMEM((ES, EB), jnp.int32),
        pltpu.VMEM((NB, EB, C), jnp.float32),
        pltpu.VMEM((RPT, C), jnp.float32),
        pltpu.VMEM_SHARED((NPAD, C), jnp.float32),
        pltpu.SemaphoreType.DMA,
        pltpu.SemaphoreType.DMA,
    ],
)
def _sc_aggregate(tab_hbm, src_hbm, dst_hbm, out_hbm,
                  si_v, di_v, rows_v, zb_v, acc, gsem, ssem):
    c = lax.axis_index("c")
    s = lax.axis_index("s")
    w = c * NS + s

    def fill(i, _):
        zb_v[i] = jnp.zeros((C,), jnp.float32)
        return 0

    lax.fori_loop(0, RPT, fill, 0)
    pltpu.sync_copy(zb_v, acc.at[pl.ds(s * RPT, RPT)])
    pltpu.sync_copy(src_hbm.at[w], si_v)
    pltpu.sync_copy(dst_hbm.at[w], di_v)
    plsc.subcore_barrier()

    for p in range(NB - 1):
        pltpu.async_copy(tab_hbm.at[si_v.at[p]], rows_v.at[p], gsem)

    def step(j, _):
        b = lax.rem(j, NB)
        pltpu.make_async_copy(tab_hbm.at[si_v.at[j]], rows_v.at[b],
                              gsem).wait()
        pltpu.async_copy(rows_v.at[b], acc.at[di_v.at[j]], ssem, add=True)

        @pl.when(j >= 1)
        def _():
            bp = lax.rem(j - 1, NB)
            pltpu.make_async_copy(rows_v.at[bp], acc.at[di_v.at[j - 1]],
                                  ssem).wait()

        @pl.when(j + NB - 1 < ES)
        def _():
            bn = lax.rem(j + NB - 1, NB)
            pltpu.async_copy(tab_hbm.at[si_v.at[j + NB - 1]], rows_v.at[bn],
                             gsem)

        return 0

    lax.fori_loop(0, ES, step, 0)
    pltpu.make_async_copy(rows_v.at[(ES - 1) % NB], acc.at[di_v.at[ES - 1]],
                          ssem).wait()
    plsc.subcore_barrier()
    pltpu.sync_copy(acc.at[pl.ds(s * RPT, RPT)],
                    out_hbm.at[c, pl.ds(s * RPT, RPT)])


def _tc_prep_body(x_ref, w1_ref, degp_ref, xs_ref, dis_ref):
    deg = degp_ref[:N, 0:1] + degp_ref[:N, 1:2] + 1.0
    dis = lax.rsqrt(deg)
    xw = jnp.dot(x_ref[...], w1_ref[...],
                 preferred_element_type=jnp.float32)
    xs = xw * dis
    xs_ref[...] = jnp.concatenate(
        [xs, jnp.zeros((N, C - H), jnp.float32)], axis=1)
    dis_ref[...] = dis


_tc_prep = pl.pallas_call(
    _tc_prep_body,
    out_shape=(jax.ShapeDtypeStruct((N, C), jnp.float32),
               jax.ShapeDtypeStruct((N, 1), jnp.float32)),
)


def _tc_mid_body(aggp_ref, xs_ref, dis_ref, b1_ref, w2_ref, out_ref):
    agg = aggp_ref[0, :N, :] + aggp_ref[1, :N, :] + xs_ref[...]
    t = dis_ref[...] * agg
    h = jnp.maximum(t[:, :H] + b1_ref[...], 0.0)
    hw = jnp.dot(h, w2_ref[...], preferred_element_type=jnp.float32)
    out_ref[...] = hw * dis_ref[...]


_tc_mid = pl.pallas_call(
    _tc_mid_body,
    out_shape=jax.ShapeDtypeStruct((N, C), jnp.float32),
)


def _tc_out_body(aggp_ref, xs2_ref, dis_ref, b2_ref, out_ref):
    o = dis_ref[...] * (aggp_ref[0, :N, :] + aggp_ref[1, :N, :]
                        + xs2_ref[...]) + b2_ref[...]
    m = jnp.max(o, axis=1, keepdims=True)
    e = jnp.exp(o - m)
    lse = jnp.log(jnp.sum(e, axis=1, keepdims=True)) + m
    out_ref[...] = o - lse


_tc_out = pl.pallas_call(
    _tc_out_body,
    out_shape=jax.ShapeDtypeStruct((N, C), jnp.float32),
)


def kernel(x, edge_index, W1, b1, W2, b2):
    src = edge_index[0].reshape(NW, ES, EB)
    dst = edge_index[1].reshape(NW, ES, EB)
    degp = _sc_degree(dst)
    xs1, dis = _tc_prep(x, W1, degp.T)
    agg1 = _sc_aggregate(xs1, src, dst)
    xs2 = _tc_mid(agg1, xs1, dis, b1.reshape(1, H), W2)
    agg2 = _sc_aggregate(xs2, src, dst)
    return _tc_out(agg2, xs2, dis, b2.reshape(1, C))


# speedup vs baseline: 52.0028x; 1.1309x over previous
import functools

import numpy as np

import jax
import jax.numpy as jnp
from jax import lax
from jax.experimental import pallas as pl
from jax.experimental.pallas import tpu as pltpu
from jax.experimental.pallas import tpu_sc as plsc

N = 10000
E = 320000
D = 128
H = 8
C = 16

NC = 2
NS = 16
NW = NC * NS
EB = 128
ES = 79
EPW = ES * EB
EPAD = NW * EPW
NPAD = 10240
RPT = NPAD // NS
NB = 4

_PAD_E = EPAD - E
_PAD_SRC = np.arange(_PAD_E, dtype=np.int32) % N
_PAD_DST = (N + np.arange(_PAD_E, dtype=np.int32) % (NPAD - N)).astype(np.int32)

_mesh = plsc.VectorSubcoreMesh(core_axis_name="c", subcore_axis_name="s")
_sc_params = pltpu.CompilerParams(use_tc_tiling_on_sc=False)


@functools.partial(
    pl.kernel,
    out_type=jax.ShapeDtypeStruct((NC, NPAD), jnp.float32),
    mesh=_mesh,
    compiler_params=_sc_params,
    scratch_types=[
        pltpu.VMEM((ES, EB), jnp.int32),
        pltpu.VMEM((EB,), jnp.float32),
        pltpu.VMEM_SHARED((NPAD,), jnp.float32),
        pltpu.SemaphoreType.DMA,
    ],
)
def _sc_degree(dst_hbm, zeros_hbm, ones_hbm, out_hbm, di_v, ones_v, acc, sem):
    c = lax.axis_index("c")
    s = lax.axis_index("s")
    w = c * NS + s
    pltpu.sync_copy(ones_hbm, ones_v)
    pltpu.sync_copy(zeros_hbm.at[pl.ds(s * RPT, RPT)],
                    acc.at[pl.ds(s * RPT, RPT)])
    pltpu.sync_copy(dst_hbm.at[w], di_v)
    plsc.subcore_barrier()

    def fire(j, _):
        pltpu.async_copy(ones_v, acc.at[di_v.at[j]], sem, add=True)
        return 0

    lax.fori_loop(0, ES, fire, 0)

    def drain(j, _):
        pltpu.make_async_copy(ones_v, acc.at[di_v.at[0]], sem).wait()
        return 0

    lax.fori_loop(0, ES, drain, 0)
    plsc.subcore_barrier()
    pltpu.sync_copy(acc.at[pl.ds(s * RPT, RPT)],
                    out_hbm.at[c, pl.ds(s * RPT, RPT)])


def _make_sc_aggregate(F):
    @functools.partial(
        pl.kernel,
        out_type=jax.ShapeDtypeStruct((NC, NPAD, F), jnp.float32),
        mesh=_mesh,
        compiler_params=_sc_params,
        scratch_types=[
            pltpu.VMEM((ES, EB), jnp.int32),
            pltpu.VMEM((ES, EB), jnp.int32),
            pltpu.VMEM((NB, EB, F), jnp.float32),
            pltpu.VMEM_SHARED((NPAD, F), jnp.float32),
            pltpu.SemaphoreType.DMA,
            pltpu.SemaphoreType.DMA,
        ],
    )
    def agg(tab_hbm, src_hbm, dst_hbm, zeros_hbm, out_hbm,
            si_v, di_v, rows_v, acc, gsem, ssem):
        c = lax.axis_index("c")
        s = lax.axis_index("s")
        w = c * NS + s
        pltpu.sync_copy(zeros_hbm.at[pl.ds(s * RPT, RPT)],
                        acc.at[pl.ds(s * RPT, RPT)])
        pltpu.sync_copy(src_hbm.at[w], si_v)
        pltpu.sync_copy(dst_hbm.at[w], di_v)
        plsc.subcore_barrier()

        for p in range(NB - 1):
            pltpu.async_copy(tab_hbm.at[si_v.at[p]], rows_v.at[p], gsem)

        def step(j, _):
            b = lax.rem(j, NB)
            pltpu.make_async_copy(tab_hbm.at[si_v.at[j]], rows_v.at[b],
                                  gsem).wait()
            pltpu.async_copy(rows_v.at[b], acc.at[di_v.at[j]], ssem, add=True)

            @pl.when(j >= 1)
            def _():
                bp = lax.rem(j - 1, NB)
                pltpu.make_async_copy(rows_v.at[bp], acc.at[di_v.at[j - 1]],
                                      ssem).wait()

            @pl.when(j + NB - 1 < ES)
            def _():
                bn = lax.rem(j + NB - 1, NB)
                pltpu.async_copy(tab_hbm.at[si_v.at[j + NB - 1]],
                                 rows_v.at[bn], gsem)

            return 0

        lax.fori_loop(0, ES, step, 0)
        pltpu.make_async_copy(rows_v.at[(ES - 1) % NB],
                              acc.at[di_v.at[ES - 1]], ssem).wait()
        plsc.subcore_barrier()
        pltpu.sync_copy(acc.at[pl.ds(s * RPT, RPT)],
                        out_hbm.at[c, pl.ds(s * RPT, RPT)])

    return agg


_sc_agg8 = _make_sc_aggregate(H)
_sc_agg16 = _make_sc_aggregate(C)


def _tc_prep_body(x_ref, w1_ref, degp_ref, xs_ref, dis_ref):
    deg = degp_ref[:N, 0:1] + degp_ref[:N, 1:2] + 1.0
    dis = lax.rsqrt(deg)
    xw = jnp.dot(x_ref[...], w1_ref[...],
                 preferred_element_type=jnp.float32)
    xs_ref[...] = xw * dis
    dis_ref[...] = dis


_tc_prep = pl.pallas_call(
    _tc_prep_body,
    out_shape=(jax.ShapeDtypeStruct((N, H), jnp.float32),
               jax.ShapeDtypeStruct((N, 1), jnp.float32)),
)


def _tc_mid_body(aggp_ref, xs_ref, dis_ref, b1_ref, w2_ref, out_ref):
    agg = aggp_ref[0, :N, :] + aggp_ref[1, :N, :] + xs_ref[...]
    h = jnp.maximum(dis_ref[...] * agg + b1_ref[...], 0.0)
    hw = jnp.dot(h, w2_ref[...], preferred_element_type=jnp.float32)
    out_ref[...] = hw * dis_ref[...]


_tc_mid = pl.pallas_call(
    _tc_mid_body,
    out_shape=jax.ShapeDtypeStruct((N, C), jnp.float32),
)


def _tc_out_body(aggp_ref, xs2_ref, dis_ref, b2_ref, out_ref):
    o = dis_ref[...] * (aggp_ref[0, :N, :] + aggp_ref[1, :N, :]
                        + xs2_ref[...]) + b2_ref[...]
    m = jnp.max(o, axis=1, keepdims=True)
    e = jnp.exp(o - m)
    lse = jnp.log(jnp.sum(e, axis=1, keepdims=True)) + m
    out_ref[...] = o - lse


_tc_out = pl.pallas_call(
    _tc_out_body,
    out_shape=jax.ShapeDtypeStruct((N, C), jnp.float32),
)


def kernel(x, edge_index, W1, b1, W2, b2):
    src = jnp.concatenate([edge_index[0], jnp.asarray(_PAD_SRC)])
    dst = jnp.concatenate([edge_index[1], jnp.asarray(_PAD_DST)])
    src = src.reshape(NW, ES, EB)
    dst = dst.reshape(NW, ES, EB)
    zeros8 = jnp.zeros((NPAD, H), jnp.float32)
    zeros16 = jnp.zeros((NPAD, C), jnp.float32)
    zeros1 = jnp.zeros((NPAD,), jnp.float32)
    ones = jnp.ones((EB,), jnp.float32)

    degp = _sc_degree(dst, zeros1, ones)
    xs1, dis = _tc_prep(x, W1, degp.T)
    agg1 = _sc_agg8(xs1, src, dst, zeros8)
    xs2 = _tc_mid(agg1, xs1, dis, b1.reshape(1, H), W2)
    agg2 = _sc_agg16(xs2, src, dst, zeros16)
    return _tc_out(agg2, xs2, dis, b2.reshape(1, C))


# speedup vs baseline: 70.1442x; 1.3489x over previous
import functools

import numpy as np

import jax
import jax.numpy as jnp
from jax import lax
from jax.experimental import pallas as pl
from jax.experimental.pallas import tpu as pltpu
from jax.experimental.pallas import tpu_sc as plsc

N = 10000
E = 320000
D = 128
H = 8
C = 16

NC = 2
NS = 16
NW = NC * NS
EB = 128
ES = 79
EPW = ES * EB
EPAD = NW * EPW
NPAD = 10240
RPT = NPAD // NS
NB = 4

_PAD_E = EPAD - E
_PAD_SRC = np.arange(_PAD_E, dtype=np.int32) % N
_PAD_DST = (N + np.arange(_PAD_E, dtype=np.int32) % (NPAD - N)).astype(np.int32)

_mesh = plsc.VectorSubcoreMesh(core_axis_name="c", subcore_axis_name="s")
_sc_params = pltpu.CompilerParams(use_tc_tiling_on_sc=False)


@functools.partial(
    pl.kernel,
    out_type=jax.ShapeDtypeStruct((NC, NPAD), jnp.float32),
    mesh=_mesh,
    compiler_params=_sc_params,
    scratch_types=[
        pltpu.VMEM((ES, EB), jnp.int32),
        pltpu.VMEM((EB,), jnp.float32),
        pltpu.VMEM_SHARED((NPAD,), jnp.float32),
        pltpu.SemaphoreType.DMA,
    ],
)
def _sc_degree(dst_hbm, zeros_hbm, ones_hbm, out_hbm, di_v, ones_v, acc, sem):
    c = lax.axis_index("c")
    s = lax.axis_index("s")
    w = c * NS + s
    pltpu.sync_copy(ones_hbm, ones_v)
    pltpu.sync_copy(zeros_hbm.at[pl.ds(s * RPT, RPT)],
                    acc.at[pl.ds(s * RPT, RPT)])
    pltpu.sync_copy(dst_hbm.at[w], di_v)
    plsc.subcore_barrier()

    def fire(j, _):
        pltpu.async_copy(ones_v, acc.at[di_v.at[j]], sem, add=True)
        return 0

    lax.fori_loop(0, ES, fire, 0)

    def drain(j, _):
        pltpu.make_async_copy(ones_v, acc.at[di_v.at[0]], sem).wait()
        return 0

    lax.fori_loop(0, ES, drain, 0)
    plsc.subcore_barrier()
    pltpu.sync_copy(acc.at[pl.ds(s * RPT, RPT)],
                    out_hbm.at[c, pl.ds(s * RPT, RPT)])


def _make_sc_aggregate(F):
    @functools.partial(
        pl.kernel,
        out_type=jax.ShapeDtypeStruct((NC, NPAD, F), jnp.float32),
        mesh=_mesh,
        compiler_params=_sc_params,
        scratch_types=[
            pltpu.VMEM((ES, EB), jnp.int32),
            pltpu.VMEM((ES, EB), jnp.int32),
            pltpu.VMEM((NB, EB, F), jnp.float32),
            pltpu.VMEM_SHARED((NPAD, F), jnp.float32),
            pltpu.VMEM_SHARED((N, F), jnp.float32),
            pltpu.SemaphoreType.DMA,
            pltpu.SemaphoreType.DMA,
        ],
    )
    def agg(tab_hbm, src_hbm, dst_hbm, zeros_hbm, out_hbm,
            si_v, di_v, rows_v, acc, tab_sh, gsem, ssem):
        c = lax.axis_index("c")
        s = lax.axis_index("s")
        w = c * NS + s
        TPT = N // NS
        pltpu.sync_copy(zeros_hbm.at[pl.ds(s * RPT, RPT)],
                        acc.at[pl.ds(s * RPT, RPT)])
        pltpu.sync_copy(tab_hbm.at[pl.ds(s * TPT, TPT)],
                        tab_sh.at[pl.ds(s * TPT, TPT)])
        pltpu.sync_copy(src_hbm.at[w], si_v)
        pltpu.sync_copy(dst_hbm.at[w], di_v)
        plsc.subcore_barrier()

        for p in range(NB - 1):
            pltpu.async_copy(tab_sh.at[si_v.at[p]], rows_v.at[p], gsem)

        def step(j, _):
            b = lax.rem(j, NB)
            pltpu.make_async_copy(tab_sh.at[si_v.at[j]], rows_v.at[b],
                                  gsem).wait()
            pltpu.async_copy(rows_v.at[b], acc.at[di_v.at[j]], ssem, add=True)

            @pl.when(j >= 1)
            def _():
                bp = lax.rem(j - 1, NB)
                pltpu.make_async_copy(rows_v.at[bp], acc.at[di_v.at[j - 1]],
                                      ssem).wait()

            @pl.when(j + NB - 1 < ES)
            def _():
                bn = lax.rem(j + NB - 1, NB)
                pltpu.async_copy(tab_sh.at[si_v.at[j + NB - 1]],
                                 rows_v.at[bn], gsem)

            return 0

        lax.fori_loop(0, ES, step, 0)
        pltpu.make_async_copy(rows_v.at[(ES - 1) % NB],
                              acc.at[di_v.at[ES - 1]], ssem).wait()
        plsc.subcore_barrier()
        pltpu.sync_copy(acc.at[pl.ds(s * RPT, RPT)],
                        out_hbm.at[c, pl.ds(s * RPT, RPT)])

    return agg


_sc_agg8 = _make_sc_aggregate(H)
_sc_agg16 = _make_sc_aggregate(C)


def _tc_prep_body(x_ref, w1_ref, degp_ref, xs_ref, dis_ref):
    deg = degp_ref[:N, 0:1] + degp_ref[:N, 1:2] + 1.0
    dis = lax.rsqrt(deg)
    xw = jnp.dot(x_ref[...], w1_ref[...],
                 preferred_element_type=jnp.float32)
    xs_ref[...] = xw * dis
    dis_ref[...] = dis


_tc_prep = pl.pallas_call(
    _tc_prep_body,
    out_shape=(jax.ShapeDtypeStruct((N, H), jnp.float32),
               jax.ShapeDtypeStruct((N, 1), jnp.float32)),
)


def _tc_mid_body(aggp_ref, xs_ref, dis_ref, b1_ref, w2_ref, out_ref):
    agg = aggp_ref[0, :N, :] + aggp_ref[1, :N, :] + xs_ref[...]
    h = jnp.maximum(dis_ref[...] * agg + b1_ref[...], 0.0)
    hw = jnp.dot(h, w2_ref[...], preferred_element_type=jnp.float32)
    out_ref[...] = hw * dis_ref[...]


_tc_mid = pl.pallas_call(
    _tc_mid_body,
    out_shape=jax.ShapeDtypeStruct((N, C), jnp.float32),
)


def _tc_out_body(aggp_ref, xs2_ref, dis_ref, b2_ref, out_ref):
    o = dis_ref[...] * (aggp_ref[0, :N, :] + aggp_ref[1, :N, :]
                        + xs2_ref[...]) + b2_ref[...]
    m = jnp.max(o, axis=1, keepdims=True)
    e = jnp.exp(o - m)
    lse = jnp.log(jnp.sum(e, axis=1, keepdims=True)) + m
    out_ref[...] = o - lse


_tc_out = pl.pallas_call(
    _tc_out_body,
    out_shape=jax.ShapeDtypeStruct((N, C), jnp.float32),
)


def kernel(x, edge_index, W1, b1, W2, b2):
    src = jnp.concatenate([edge_index[0], jnp.asarray(_PAD_SRC)])
    dst = jnp.concatenate([edge_index[1], jnp.asarray(_PAD_DST)])
    src = src.reshape(NW, ES, EB)
    dst = dst.reshape(NW, ES, EB)
    zeros8 = jnp.zeros((NPAD, H), jnp.float32)
    zeros16 = jnp.zeros((NPAD, C), jnp.float32)
    zeros1 = jnp.zeros((NPAD,), jnp.float32)
    ones = jnp.ones((EB,), jnp.float32)

    degp = _sc_degree(dst, zeros1, ones)
    xs1, dis = _tc_prep(x, W1, degp.T)
    agg1 = _sc_agg8(xs1, src, dst, zeros8)
    xs2 = _tc_mid(agg1, xs1, dis, b1.reshape(1, H), W2)
    agg2 = _sc_agg16(xs2, src, dst, zeros16)
    return _tc_out(agg2, xs2, dis, b2.reshape(1, C))
